# Initial kernel scaffold; baseline (speedup 1.0000x reference)
import functools

import jax
import jax.numpy as jnp
from jax import lax
from jax.experimental import pallas as pl
from jax.experimental.pallas import tpu as pltpu
from jax.experimental.pallas import tpu_sc as plsc

VOCAB = 100000
DIM = 64
B = 16384
K = 20

NC = 2
NS = 16
NW = NC * NS
BPW = B // NW
CB = 64
NCHUNK = BPW // CB


def _sc_body(tw_hbm, cw_hbm, nw_hbm, tt_hbm, ct_hbm,
             pos_hbm, neg_hbm,
             ti_v, ci_v, ni_v, tr_v, cr_v, nr_v, po_v, no_v, sem):
    wid = lax.axis_index("s") * NC + lax.axis_index("c")

    def chunk(ci, carry):
        base = wid * BPW + ci * CB
        pltpu.sync_copy(tw_hbm.at[pl.ds(base, CB)], ti_v)
        pltpu.sync_copy(cw_hbm.at[pl.ds(base, CB)], ci_v)
        pltpu.sync_copy(nw_hbm.at[pl.ds(base * K, CB * K)], ni_v)
        cp1 = pltpu.async_copy(tt_hbm.at[ti_v], tr_v, sem)
        cp2 = pltpu.async_copy(ct_hbm.at[ci_v], cr_v, sem)
        cp3 = pltpu.async_copy(ct_hbm.at[ni_v], nr_v, sem)
        cp1.wait()
        cp2.wait()
        cp3.wait()

        def body(b, carry2):
            t = [tr_v[b, pl.ds(16 * j, 16)] for j in range(4)]
            c = [cr_v[b, pl.ds(16 * j, 16)] for j in range(4)]
            p = t[0] * c[0] + t[1] * c[1] + t[2] * c[2] + t[3] * c[3]
            ps = jnp.sum(p)
            po_v[b] = jnp.clip(ps, -10.0, 10.0)
            for k in range(K):
                r = b * K + k
                q = (nr_v[r, pl.ds(0, 16)] * t[0]
                     + nr_v[r, pl.ds(16, 16)] * t[1]
                     + nr_v[r, pl.ds(32, 16)] * t[2]
                     + nr_v[r, pl.ds(48, 16)] * t[3])
                qs = jnp.sum(q)
                no_v[r] = jnp.clip(qs, -10.0, 10.0)
            return carry2

        lax.fori_loop(0, CB, body, 0)
        pltpu.sync_copy(po_v, pos_hbm.at[pl.ds(base, CB)])
        pltpu.sync_copy(no_v, neg_hbm.at[pl.ds(base * K, CB * K)])
        return carry

    lax.fori_loop(0, NCHUNK, chunk, 0)


_sc_call = functools.partial(
    pl.kernel,
    out_type=[
        jax.ShapeDtypeStruct((B,), jnp.float32),
        jax.ShapeDtypeStruct((B * K,), jnp.float32),
    ],
    mesh=plsc.VectorSubcoreMesh(core_axis_name="c", subcore_axis_name="s"),
    scratch_types=[
        pltpu.VMEM((CB,), jnp.int32),
        pltpu.VMEM((CB,), jnp.int32),
        pltpu.VMEM((CB * K,), jnp.int32),
        pltpu.VMEM((CB, DIM), jnp.float32),
        pltpu.VMEM((CB, DIM), jnp.float32),
        pltpu.VMEM((CB * K, DIM), jnp.float32),
        pltpu.VMEM((CB,), jnp.float32),
        pltpu.VMEM((CB * K,), jnp.float32),
        pltpu.SemaphoreType.DMA,
    ],
)(_sc_body)


def kernel(target_word, context_word, negative_words, target_table, context_table):
    neg_flat = negative_words.reshape(-1).astype(jnp.int32)
    pos, neg = _sc_call(
        target_word.astype(jnp.int32),
        context_word.astype(jnp.int32),
        neg_flat,
        target_table,
        context_table,
    )
    return pos, neg.reshape(B, K)


# speedup vs baseline: 4.2741x; 4.2741x over previous
import functools

import jax
import jax.numpy as jnp
from jax import lax
from jax.experimental import pallas as pl
from jax.experimental.pallas import tpu as pltpu
from jax.experimental.pallas import tpu_sc as plsc

VOCAB = 100000
DIM = 64
B = 16384
K = 20

NC = 2
NS = 16
NW = NC * NS
BPW = B // NW
CB = 64
NCHUNK = BPW // CB


def _sc_body(tw_hbm, cw_hbm, nw_hbm, tt_hbm, ct_hbm,
             pos_hbm, neg_hbm,
             ti_v, ci_v, ni_v, tr_v, cr_v, nr_v, po_v, no_v, tp_v, sem):
    wid = lax.axis_index("s") * NC + lax.axis_index("c")

    def chunk(ci, carry):
        base = wid * BPW + ci * CB
        pltpu.sync_copy(tw_hbm.at[pl.ds(base, CB)], ti_v)
        pltpu.sync_copy(cw_hbm.at[pl.ds(base, CB)], ci_v)
        pltpu.sync_copy(nw_hbm.at[pl.ds(base * K, CB * K)], ni_v)
        cp1 = pltpu.async_copy(tt_hbm.at[ti_v], tr_v, sem)
        cp2 = pltpu.async_copy(ct_hbm.at[ci_v], cr_v, sem)
        cp3 = pltpu.async_copy(ct_hbm.at[ni_v], nr_v, sem)
        cp1.wait()
        cp2.wait()
        cp3.wait()

        lanes = lax.iota(jnp.int32, 16)

        def prods(row_ref, r, t):
            return (row_ref[r, pl.ds(0, 16)] * t[0]
                    + row_ref[r, pl.ds(16, 16)] * t[1]
                    + row_ref[r, pl.ds(32, 16)] * t[2]
                    + row_ref[r, pl.ds(48, 16)] * t[3])

        def hsum16(base):
            acc = plsc.load_gather(tp_v, [base + lanes * 16])
            for j in range(1, 16):
                acc = acc + plsc.load_gather(tp_v, [base + lanes * 16 + j])
            return acc

        def body(b, carry2):
            t = [tr_v[b, pl.ds(16 * j, 16)] for j in range(4)]
            for k in range(16):
                tp_v[pl.ds(k * 16, 16)] = prods(nr_v, b * K + k, t)
            for k in range(16, K):
                tp_v[pl.ds(256 + (k - 16) * 16, 16)] = prods(nr_v, b * K + k, t)
            tp_v[pl.ds(256 + 64, 16)] = prods(cr_v, b, t)
            sa = jnp.clip(hsum16(0), -10.0, 10.0)
            plsc.store_scatter(no_v, [b * K + lanes], sa)
            sb = jnp.clip(hsum16(256), -10.0, 10.0)
            plsc.store_scatter(no_v, [b * K + 16 + lanes], sb,
                               mask=lanes < (K - 16))
            plsc.store_scatter(po_v, [jnp.full((16,), b, jnp.int32)], sb,
                               mask=lanes == (K - 16))
            return carry2

        lax.fori_loop(0, CB, body, 0)
        pltpu.sync_copy(po_v, pos_hbm.at[pl.ds(base, CB)])
        pltpu.sync_copy(no_v.at[pl.ds(0, CB * K)],
                        neg_hbm.at[pl.ds(base * K, CB * K)])
        return carry

    lax.fori_loop(0, NCHUNK, chunk, 0)


_sc_call = functools.partial(
    pl.kernel,
    out_type=[
        jax.ShapeDtypeStruct((B,), jnp.float32),
        jax.ShapeDtypeStruct((B * K,), jnp.float32),
    ],
    mesh=plsc.VectorSubcoreMesh(core_axis_name="c", subcore_axis_name="s"),
    compiler_params=pltpu.CompilerParams(needs_layout_passes=False,
                                         use_tc_tiling_on_sc=False),
    scratch_types=[
        pltpu.VMEM((CB,), jnp.int32),
        pltpu.VMEM((CB,), jnp.int32),
        pltpu.VMEM((CB * K,), jnp.int32),
        pltpu.VMEM((CB, DIM), jnp.float32),
        pltpu.VMEM((CB, DIM), jnp.float32),
        pltpu.VMEM((CB * K, DIM), jnp.float32),
        pltpu.VMEM((CB,), jnp.float32),
        pltpu.VMEM((CB * K + 16,), jnp.float32),
        pltpu.VMEM((512,), jnp.float32),
        pltpu.SemaphoreType.DMA,
    ],
)(_sc_body)


def kernel(target_word, context_word, negative_words, target_table, context_table):
    neg_flat = negative_words.reshape(-1).astype(jnp.int32)
    pos, neg = _sc_call(
        target_word.astype(jnp.int32),
        context_word.astype(jnp.int32),
        neg_flat,
        target_table,
        context_table,
    )
    return pos, neg.reshape(B, K)


# speedup vs baseline: 4.8595x; 1.1370x over previous
import functools

import jax
import jax.numpy as jnp
from jax import lax
from jax.experimental import pallas as pl
from jax.experimental.pallas import tpu as pltpu
from jax.experimental.pallas import tpu_sc as plsc

VOCAB = 100000
DIM = 64
B = 16384
K = 20

NC = 2
NS = 16
NW = NC * NS
BPW = B // NW
CB = 32
NCHUNK = BPW // CB


def _sc_body(tw_hbm, cw_hbm, nw_hbm, tt_hbm, ct_hbm,
             pos_hbm, neg_hbm,
             ti_v, ci_v, ni_v, po_v, no_v, tp_v,
             tr0, cr0, nr0, tr1, cr1, nr1, sem0, sem1):
    wid = lax.axis_index("s") * NC + lax.axis_index("c")
    base = wid * BPW
    pltpu.sync_copy(tw_hbm.at[pl.ds(base, BPW)], ti_v)
    pltpu.sync_copy(cw_hbm.at[pl.ds(base, BPW)], ci_v)
    pltpu.sync_copy(nw_hbm.at[pl.ds(base * K, BPW * K)], ni_v)

    bufs = ((tr0, cr0, nr0, sem0), (tr1, cr1, nr1, sem1))
    lanes = lax.iota(jnp.int32, 16)

    def issue(c, slot):
        tr, cr, nr, sem = bufs[slot]
        o = c * CB
        pltpu.async_copy(tt_hbm.at[ti_v.at[pl.ds(o, CB)]], tr, sem)
        pltpu.async_copy(ct_hbm.at[ci_v.at[pl.ds(o, CB)]], cr, sem)
        pltpu.async_copy(ct_hbm.at[ni_v.at[pl.ds(o * K, CB * K)]], nr, sem)

    def drain(slot):
        tr, cr, nr, sem = bufs[slot]
        pltpu.make_async_copy(tt_hbm.at[pl.ds(0, CB)], tr, sem).wait()
        pltpu.make_async_copy(ct_hbm.at[pl.ds(0, CB)], cr, sem).wait()
        pltpu.make_async_copy(ct_hbm.at[pl.ds(0, CB * K)], nr, sem).wait()

    def prods(row_ref, r, t):
        return (row_ref[r, pl.ds(0, 16)] * t[0]
                + row_ref[r, pl.ds(16, 16)] * t[1]
                + row_ref[r, pl.ds(32, 16)] * t[2]
                + row_ref[r, pl.ds(48, 16)] * t[3])

    def hsum16(bb):
        acc = plsc.load_gather(tp_v, [bb + lanes * 16])
        for j in range(1, 16):
            acc = acc + plsc.load_gather(tp_v, [bb + lanes * 16 + j])
        return acc

    def compute(c, slot):
        tr, cr, nr, _ = bufs[slot]

        def body(b, carry):
            g = c * CB + b
            t = [tr[b, pl.ds(16 * j, 16)] for j in range(4)]
            for k in range(16):
                tp_v[pl.ds(k * 16, 16)] = prods(nr, b * K + k, t)
            for k in range(16, K):
                tp_v[pl.ds(256 + (k - 16) * 16, 16)] = prods(nr, b * K + k, t)
            tp_v[pl.ds(256 + 64, 16)] = prods(cr, b, t)
            sa = jnp.clip(hsum16(0), -10.0, 10.0)
            plsc.store_scatter(no_v, [g * K + lanes], sa)
            sb = jnp.clip(hsum16(256), -10.0, 10.0)
            plsc.store_scatter(no_v, [g * K + 16 + lanes], sb,
                               mask=lanes < (K - 16))
            plsc.store_scatter(po_v, [jnp.full((16,), g, jnp.int32)], sb,
                               mask=lanes == (K - 16))
            return carry

        lax.fori_loop(0, CB, body, 0)

    issue(0, 0)

    def pair(i, carry):
        g = i * 2
        issue(g + 1, 1)
        drain(0)
        compute(g, 0)

        @pl.when(g + 2 < NCHUNK)
        def _():
            issue(g + 2, 0)

        drain(1)
        compute(g + 1, 1)
        return carry

    lax.fori_loop(0, NCHUNK // 2, pair, 0)

    pltpu.sync_copy(po_v, pos_hbm.at[pl.ds(base, BPW)])
    pltpu.sync_copy(no_v.at[pl.ds(0, BPW * K)],
                    neg_hbm.at[pl.ds(base * K, BPW * K)])


_sc_call = functools.partial(
    pl.kernel,
    out_type=[
        jax.ShapeDtypeStruct((B,), jnp.float32),
        jax.ShapeDtypeStruct((B * K,), jnp.float32),
    ],
    mesh=plsc.VectorSubcoreMesh(core_axis_name="c", subcore_axis_name="s"),
    compiler_params=pltpu.CompilerParams(needs_layout_passes=False,
                                         use_tc_tiling_on_sc=False),
    scratch_types=[
        pltpu.VMEM((BPW,), jnp.int32),
        pltpu.VMEM((BPW,), jnp.int32),
        pltpu.VMEM((BPW * K,), jnp.int32),
        pltpu.VMEM((BPW,), jnp.float32),
        pltpu.VMEM((BPW * K + 16,), jnp.float32),
        pltpu.VMEM((512,), jnp.float32),
        pltpu.VMEM((CB, DIM), jnp.float32),
        pltpu.VMEM((CB, DIM), jnp.float32),
        pltpu.VMEM((CB * K, DIM), jnp.float32),
        pltpu.VMEM((CB, DIM), jnp.float32),
        pltpu.VMEM((CB, DIM), jnp.float32),
        pltpu.VMEM((CB * K, DIM), jnp.float32),
        pltpu.SemaphoreType.DMA,
        pltpu.SemaphoreType.DMA,
    ],
)(_sc_body)


def kernel(target_word, context_word, negative_words, target_table, context_table):
    neg_flat = negative_words.reshape(-1).astype(jnp.int32)
    pos, neg = _sc_call(
        target_word.astype(jnp.int32),
        context_word.astype(jnp.int32),
        neg_flat,
        target_table,
        context_table,
    )
    return pos, neg.reshape(B, K)
